# baseline (speedup 1.0000x reference)
import functools

import jax
import jax.numpy as jnp
from jax import lax
from jax.experimental import pallas as pl
from jax.experimental.pallas import tpu as pltpu
from jax.experimental.pallas import tpu_sc as plsc

_B, _C, _H, _W = 8, 21, 512, 512
_HW = _H * _W
_EPS = 1e-05

_G = 3
_CG = _C // _G
_ROWS = _HW // 1024
_KJ = 64
_NJ = _ROWS // _KJ


def _tc_body(pred_ref, tgt_ref, out_ref):
    g = pl.program_id(1)
    j = pl.program_id(2)
    cid = g * _CG + lax.broadcasted_iota(jnp.int32, (_CG, 1, 1), 0)

    def step(k, accs):
        ia, p2a = accs
        p = pred_ref[0, :, k, :, :]
        t = tgt_ref[0, 0, k, :, :]
        m = t[None, :, :] == cid
        ia = ia + jnp.where(m, p, 0.0)
        p2a = p2a + p * p
        return ia, p2a

    zero = jnp.zeros((_CG, 8, 128), jnp.float32)
    ia, p2a = lax.fori_loop(0, _KJ, step, (zero, zero), unroll=2)
    part = jnp.concatenate([ia, p2a], axis=0)
    part = part.reshape(2 * _CG * 8, 128)

    @pl.when(j == 0)
    def _():
        out_ref[0, 0] = part

    @pl.when(j != 0)
    def _():
        out_ref[0, 0] += part


_NC, _NS = 2, 16
_NW = _NC * _NS
_PXW = (_B * _HW) // _NW
_CHUNK_SC = 1024
_NCH = _PXW // _CHUNK_SC


def _sc_hist_body(tgt_hbm, out_hbm, tbuf, bins):
    wid = lax.axis_index("s") * _NC + lax.axis_index("c")
    base = wid * _PXW
    ones = jnp.ones((16,), jnp.float32)
    zeros = jnp.zeros((16,), jnp.float32)
    li = lax.iota(jnp.int32, 16)

    for r in range(_C):
        bins[pl.ds(r * 16, 16)] = zeros

    def chunk(i, carry):
        pltpu.sync_copy(tgt_hbm.at[pl.ds(base + i * _CHUNK_SC, _CHUNK_SC)],
                        tbuf)
        for s in range(_CHUNK_SC // 16):
            tv = tbuf[pl.ds(s * 16, 16)]
            plsc.addupdate_scatter(bins, [tv * 16 + li], ones)
        return carry

    lax.fori_loop(0, _NCH, chunk, 0)
    pltpu.sync_copy(bins, out_hbm.at[wid])


def _sc_hist(tgt_flat):
    mesh = plsc.VectorSubcoreMesh(core_axis_name="c", subcore_axis_name="s")
    return pl.kernel(
        _sc_hist_body,
        out_type=jax.ShapeDtypeStruct((_NW, _C * 16), jnp.float32),
        mesh=mesh,
        scratch_types=[
            pltpu.VMEM((_CHUNK_SC,), jnp.int32),
            pltpu.VMEM((_C * 16,), jnp.float32),
        ],
        compiler_params=pltpu.CompilerParams(needs_layout_passes=False),
    )(tgt_flat)


def kernel(prediction, target):
    tgt = target.astype(jnp.int32)
    pred5 = prediction.reshape(_B * _G, _CG, _ROWS, 8, 128)
    tgt5 = tgt.reshape(_B, 1, _ROWS, 8, 128)

    sums = pl.pallas_call(
        _tc_body,
        grid=(_B, _G, _NJ),
        in_specs=[
            pl.BlockSpec((1, _CG, _KJ, 8, 128),
                         lambda b, g, j: (b * _G + g, 0, j, 0, 0)),
            pl.BlockSpec((1, 1, _KJ, 8, 128),
                         lambda b, g, j: (b, 0, j, 0, 0)),
        ],
        out_specs=pl.BlockSpec((1, 1, 2 * _CG * 8, 128),
                               lambda b, g, j: (b, g, 0, 0)),
        out_shape=jax.ShapeDtypeStruct((_B, _G, 2 * _CG * 8, 128),
                                       jnp.float32),
    )(pred5, tgt5)

    hist = _sc_hist(tgt.reshape(_B * _HW))

    s = sums.reshape(_B, _G, 2, _CG, 8, 128).sum(axis=(4, 5))
    s = s.transpose(0, 2, 1, 3).reshape(_B, 2, _C)
    inter, p2 = s[:, 0], s[:, 1]
    cnt = hist.reshape(_B, _NW // _B, _C, 16).sum(axis=(1, 3))
    dice = (2.0 * inter + _EPS) / (p2 + cnt + _EPS)
    return 1.0 - dice.mean()
